# baseline (speedup 1.0000x reference)
import functools

import jax
import jax.numpy as jnp
from jax import lax
from jax.experimental import pallas as pl
from jax.experimental.pallas import tpu as pltpu
from jax.experimental.pallas import tpu_sc as plsc

N_NODES = 10000
N_EDGES = 320000
H = 128
S = 16
F = 16
W_ATT = 272
DPG = 384
DPE = 288
L = 16
NW = 32
CHUNK = 128
SUB = 32
NSUB = CHUNK // SUB
NCH_TOTAL = N_EDGES // CHUNK
DENOM_STRIDE = 10240
STRIPE = 1000

_mesh = plsc.VectorSubcoreMesh(core_axis_name="c", subcore_axis_name="s")
_sc_params = pltpu.CompilerParams(needs_layout_passes=False)


def _node_prep_body(xc_ref, ws_ref, wd_ref, b1_ref, as_ref, ad_ref):
    xc = xc_ref[...]
    as_ref[...] = jnp.dot(xc, ws_ref[...], preferred_element_type=jnp.float32)
    ad_ref[...] = jnp.dot(
        xc, wd_ref[...], preferred_element_type=jnp.float32) + b1_ref[...]


def _node_prep(xcat, wsT, wdT, b1p):
    blk = 2000
    grid = N_NODES // blk
    return pl.pallas_call(
        _node_prep_body,
        grid=(grid,),
        in_specs=[
            pl.BlockSpec((blk, H + S), lambda i: (i, 0)),
            pl.BlockSpec((H + S, DPG), lambda i: (0, 0)),
            pl.BlockSpec((H + S, DPG), lambda i: (0, 0)),
            pl.BlockSpec((1, DPG), lambda i: (0, 0)),
        ],
        out_specs=[
            pl.BlockSpec((blk, DPG), lambda i: (i, 0)),
            pl.BlockSpec((blk, DPG), lambda i: (i, 0)),
        ],
        out_shape=[
            jax.ShapeDtypeStruct((N_NODES, DPG), jnp.float32),
            jax.ShapeDtypeStruct((N_NODES, DPG), jnp.float32),
        ],
    )(xcat, wsT, wdT, b1p)


def _edge_prep_body(ef_ref, we_ref, e_ref):
    e_ref[...] = jnp.dot(
        ef_ref[...], we_ref[...], preferred_element_type=jnp.float32)


def _edge_prep(ef, weT):
    blk = 8000
    grid = N_EDGES // blk
    return pl.pallas_call(
        _edge_prep_body,
        grid=(grid,),
        in_specs=[
            pl.BlockSpec((blk, F), lambda i: (i, 0)),
            pl.BlockSpec((F, DPE), lambda i: (0, 0)),
        ],
        out_specs=pl.BlockSpec((blk, DPE), lambda i: (i, 0)),
        out_shape=jax.ShapeDtypeStruct((N_EDGES, DPE), jnp.float32),
    )(ef, weT)


@functools.partial(
    pl.kernel,
    out_type=[
        jax.ShapeDtypeStruct((N_EDGES,), jnp.float32),
        jax.ShapeDtypeStruct((NW * DENOM_STRIDE,), jnp.float32),
    ],
    mesh=_mesh,
    compiler_params=_sc_params,
    scratch_types=[
        pltpu.VMEM((CHUNK,), jnp.int32),
        pltpu.VMEM((CHUNK,), jnp.int32),
        pltpu.VMEM((CHUNK,), jnp.int32),
        pltpu.VMEM((CHUNK,), jnp.int32),
        [pltpu.VMEM((SUB,), jnp.int32) for _ in range(2)],
        [pltpu.VMEM((SUB,), jnp.int32) for _ in range(2)],
        pltpu.VMEM((2, SUB, DPG), jnp.float32),
        pltpu.VMEM((2, SUB, DPG), jnp.float32),
        pltpu.VMEM((2, SUB, DPE), jnp.float32),
        pltpu.VMEM((CHUNK,), jnp.float32),
        pltpu.VMEM((DENOM_STRIDE,), jnp.float32),
        pltpu.VMEM((DPE + L,), jnp.float32),
        [pltpu.SemaphoreType.DMA for _ in range(2)],
        [pltpu.SemaphoreType.DMA for _ in range(2)],
    ],
)
def _sc_phase1(asrc_hbm, adst_hbm, e_hbm, src_hbm, dst_hbm, w2_hbm,
               exp_out, denom_out,
               srcb0, dstb0, srcb1, dstb1, srcsub, dstsub,
               abufs, abufd, ebuf, expb, denom, w2v, sg, se):
    wid = lax.axis_index("s") * 2 + lax.axis_index("c")
    cnt = 78 + jnp.where(wid < 4, 1, 0)
    start = 78 * wid + jnp.minimum(wid, 4)

    pltpu.sync_copy(w2_hbm, w2v)
    b2s = jnp.sum(w2v[pl.ds(DPE, L)])
    iota16 = lax.iota(jnp.int32, L)
    w2r = [w2v[pl.ds(k * L, L)] for k in range(W_ATT // L + 1)]

    def zero_body(i, _):
        denom[pl.ds(i * L, L)] = jnp.zeros((L,), jnp.float32)
        return 0
    lax.fori_loop(0, DENOM_STRIDE // L, zero_body, 0)

    def stage_idx(b, s_dyn, from_src, from_dst):
        for i in range(SUB // L):
            sl = pl.ds(i * L, L)
            srcsub[b][sl] = from_src[pl.ds(s_dyn * SUB + i * L, L)]
            dstsub[b][sl] = from_dst[pl.ds(s_dyn * SUB + i * L, L)]

    def issue_sub(b, c_dyn, s_dyn):
        base = (start + c_dyn) * CHUNK
        pltpu.async_copy(asrc_hbm.at[srcsub[b]], abufs.at[b], sg[b])
        pltpu.async_copy(adst_hbm.at[dstsub[b]], abufd.at[b], sg[b])
        pltpu.async_copy(e_hbm.at[pl.ds(base + s_dyn * SUB, SUB)], ebuf.at[b],
                         se[b])

    base0 = start * CHUNK
    pltpu.sync_copy(src_hbm.at[pl.ds(base0, CHUNK)], srcb0)
    pltpu.sync_copy(dst_hbm.at[pl.ds(base0, CHUNK)], dstb0)
    for s_ in range(2):
        stage_idx(s_, s_, srcb0, dstb0)
        issue_sub(s_, 0, s_)

    def chunk_body(c, _):
        base = (start + c) * CHUNK
        nxt = c + 1
        has_next = nxt < cnt
        np_odd = (nxt % 2) == 1
        cur_even = (c % 2) == 0

        @pl.when(has_next & np_odd)
        def _():
            nbase = (start + nxt) * CHUNK
            pltpu.sync_copy(src_hbm.at[pl.ds(nbase, CHUNK)], srcb1)
            pltpu.sync_copy(dst_hbm.at[pl.ds(nbase, CHUNK)], dstb1)

        @pl.when(has_next & jnp.logical_not(np_odd))
        def _():
            nbase = (start + nxt) * CHUNK
            pltpu.sync_copy(src_hbm.at[pl.ds(nbase, CHUNK)], srcb0)
            pltpu.sync_copy(dst_hbm.at[pl.ds(nbase, CHUNK)], dstb0)

        for s_ in range(4):
            b = s_ % 2
            pltpu.make_async_copy(asrc_hbm.at[srcsub[b]], abufs.at[b],
                                  sg[b]).wait()
            pltpu.make_async_copy(adst_hbm.at[dstsub[b]], abufd.at[b],
                                  sg[b]).wait()
            pltpu.make_async_copy(e_hbm.at[pl.ds(base + s_ * SUB, SUB)],
                                  ebuf.at[b], se[b]).wait()

            def group_body(g, _):
                def edge_body(j, raw16):
                    e = g * L + j
                    acc = jnp.zeros((L,), jnp.float32)
                    for k in range(W_ATT // L + 1):
                        sl = pl.ds(k * L, L)
                        v = abufs[b, e, sl] + abufd[b, e, sl] + ebuf[b, e, sl]
                        v = jnp.maximum(v, 0.0)
                        acc = acc + v * w2r[k]
                    r = jnp.sum(acc)
                    return jnp.where(iota16 == j, r, raw16)

                raw16 = lax.fori_loop(0, L, edge_body,
                                      jnp.zeros((L,), jnp.float32))
                raw16 = raw16 + b2s
                raw16 = jnp.where(raw16 >= 0.0, raw16, 0.01 * raw16)
                ex16 = jnp.exp(raw16)
                expb[pl.ds(s_ * SUB + g * L, L)] = ex16

                @pl.when(cur_even)
                def _():
                    plsc.addupdate_scatter(
                        denom, [dstb0[pl.ds(s_ * SUB + g * L, L)]], ex16)

                @pl.when(jnp.logical_not(cur_even))
                def _():
                    plsc.addupdate_scatter(
                        denom, [dstb1[pl.ds(s_ * SUB + g * L, L)]], ex16)
                return 0

            lax.fori_loop(0, SUB // L, group_body, 0)

            if s_ < 2:
                @pl.when(cur_even)
                def _():
                    stage_idx(b, s_ + 2, srcb0, dstb0)

                @pl.when(jnp.logical_not(cur_even))
                def _():
                    stage_idx(b, s_ + 2, srcb1, dstb1)
                issue_sub(b, c, s_ + 2)
            else:
                @pl.when(has_next & np_odd)
                def _():
                    stage_idx(b, s_ - 2, srcb1, dstb1)
                    issue_sub(b, nxt, s_ - 2)

                @pl.when(has_next & jnp.logical_not(np_odd))
                def _():
                    stage_idx(b, s_ - 2, srcb0, dstb0)
                    issue_sub(b, nxt, s_ - 2)

        pltpu.sync_copy(expb, exp_out.at[pl.ds(base, CHUNK)])
        return 0

    lax.fori_loop(0, cnt, chunk_body, 0)
    pltpu.sync_copy(denom, denom_out.at[pl.ds(wid * DENOM_STRIDE, DENOM_STRIDE)])


@functools.partial(
    pl.kernel,
    out_type=jax.ShapeDtypeStruct((2 * N_NODES, H), jnp.float32),
    mesh=_mesh,
    compiler_params=_sc_params,
    scratch_types=[
        [pltpu.VMEM((CHUNK,), jnp.int32) for _ in range(2)],
        [pltpu.VMEM((CHUNK,), jnp.int32) for _ in range(2)],
        [pltpu.VMEM((CHUNK,), jnp.float32) for _ in range(2)],
        pltpu.VMEM((CHUNK,), jnp.float32),
        pltpu.VMEM((2, CHUNK, H), jnp.float32),
        pltpu.VMEM((DENOM_STRIDE,), jnp.float32),
        pltpu.VMEM((NW, 128), jnp.float32),
        pltpu.VMEM((8, H), jnp.float32),
        pltpu.VMEM_SHARED((DENOM_STRIDE,), jnp.float32),
        pltpu.VMEM_SHARED((N_NODES, H), jnp.float32),
        [pltpu.SemaphoreType.DMA for _ in range(2)],
        [pltpu.SemaphoreType.DMA for _ in range(2)],
    ],
)
def _sc_phase2(x_hbm, exp_hbm, denom_part_hbm, src_hbm, dst_hbm,
               agg_out,
               srcb, dstb, expb, attnb, xbuf, denom, tmp, zblk, denom_sh,
               agg_sp, sg, ss):
    cid = lax.axis_index("c")
    sid = lax.axis_index("s")
    wid = sid * 2 + cid
    cnt = 78 + jnp.where(wid < 4, 1, 0)
    start = 78 * wid + jnp.minimum(wid, 4)

    stripe0 = sid * (DENOM_STRIDE // 16)
    for q in range(DENOM_STRIDE // 16 // 128):
        pltpu.sync_copy(
            denom_part_hbm.at[:, pl.ds(stripe0 + q * 128, 128)], tmp)

        def red_body(i, _):
            sl = pl.ds(i * L, L)
            acc = tmp[0, sl]
            for p in range(1, NW):
                acc = acc + tmp[p, sl]
            denom[pl.ds(stripe0 + q * 128 + i * L, L)] = acc
            return 0
        lax.fori_loop(0, 128 // L, red_body, 0)
    pltpu.sync_copy(denom.at[pl.ds(stripe0, DENOM_STRIDE // 16)],
                    denom_sh.at[pl.ds(stripe0, DENOM_STRIDE // 16)])

    def zero_body(i, _):
        for k in range(H // L):
            zblk[i, pl.ds(k * L, L)] = jnp.zeros((L,), jnp.float32)
        return 0
    lax.fori_loop(0, 8, zero_body, 0)

    @pl.when(sid < 10)
    def _():
        def zcopy_body(r, _):
            pltpu.sync_copy(zblk, agg_sp.at[pl.ds(sid * STRIPE + r * 8, 8)])
            return 0
        lax.fori_loop(0, STRIPE // 8, zcopy_body, 0)
    plsc.subcore_barrier()
    pltpu.sync_copy(denom_sh, denom)

    def fetch(b, c_dyn):
        base = (start + c_dyn) * CHUNK
        pltpu.sync_copy(src_hbm.at[pl.ds(base, CHUNK)], srcb[b])
        pltpu.sync_copy(dst_hbm.at[pl.ds(base, CHUNK)], dstb[b])
        pltpu.sync_copy(exp_hbm.at[pl.ds(base, CHUNK)], expb[b])
        pltpu.async_copy(x_hbm.at[srcb[b]], xbuf.at[b], sg[b])

    fetch(0, 0)

    def chunk_body(c, _):
        nxt = c + 1
        for b in range(2):
            @pl.when((c % 2) == b)
            def _():
                pltpu.make_async_copy(x_hbm.at[srcb[b]], xbuf.at[b],
                                      sg[b]).wait()

                def attn_body(g, _):
                    sl = pl.ds(g * L, L)
                    den16 = plsc.load_gather(denom, [dstb[b][sl]])
                    attnb[sl] = expb[b][sl] / (den16 + 1e-09)
                    return 0
                lax.fori_loop(0, CHUNK // L, attn_body, 0)

                def scale_body(g, _):
                    a16 = attnb[pl.ds(g * L, L)]
                    for j in range(L):
                        e = g * L + j
                        aj = jnp.take(a16, jnp.full((L,), j, jnp.int32))
                        for k in range(H // L):
                            sl = pl.ds(k * L, L)
                            xbuf[b, e, sl] = xbuf[b, e, sl] * aj
                    return 0
                lax.fori_loop(0, CHUNK // L, scale_body, 0)

                pltpu.async_copy(xbuf.at[b], agg_sp.at[dstb[b]], ss[b],
                                 add=True)
                @pl.when(nxt < cnt)
                def _():
                    @pl.when(c >= 1)
                    def _():
                        pltpu.make_async_copy(
                            xbuf.at[1 - b], agg_sp.at[dstb[1 - b]],
                            ss[1 - b]).wait()
                    fetch(1 - b, nxt)
        return 0

    lax.fori_loop(0, cnt, chunk_body, 0)
    for b in range(2):
        pltpu.make_async_copy(xbuf.at[b], agg_sp.at[dstb[b]], ss[b]).wait()
    plsc.subcore_barrier()

    @pl.when(sid < 10)
    def _():
        pltpu.sync_copy(
            agg_sp.at[pl.ds(sid * STRIPE, STRIPE)],
            agg_out.at[pl.ds(cid * N_NODES + sid * STRIPE, STRIPE)])


def _update_mlp_body(x_ref, a0_ref, a1_ref, w1x_ref, w1a_ref, b1_ref, w2_ref,
                     b2_ref, out_ref):
    agg = a0_ref[...] + a1_ref[...]
    h = jnp.dot(x_ref[...], w1x_ref[...], preferred_element_type=jnp.float32)
    h += jnp.dot(agg, w1a_ref[...], preferred_element_type=jnp.float32)
    h = jax.nn.relu(h + b1_ref[...])
    o = jnp.dot(h, w2_ref[...], preferred_element_type=jnp.float32) + b2_ref[...]
    out_ref[...] = jax.nn.relu(o)


def _update_mlp(x, agg2, upd_W1, upd_b1, upd_W2, upd_b2):
    blk = 2000
    grid = N_NODES // blk
    w1x = upd_W1[:, :H].T
    w1a = upd_W1[:, H:].T
    w2 = upd_W2.T
    b1 = upd_b1[None, :]
    b2 = upd_b2[None, :]
    return pl.pallas_call(
        _update_mlp_body,
        grid=(grid,),
        in_specs=[
            pl.BlockSpec((blk, H), lambda i: (i, 0)),
            pl.BlockSpec((blk, H), lambda i: (i, 0)),
            pl.BlockSpec((blk, H), lambda i: (i + grid, 0)),
            pl.BlockSpec((H, 2 * H), lambda i: (0, 0)),
            pl.BlockSpec((H, 2 * H), lambda i: (0, 0)),
            pl.BlockSpec((1, 2 * H), lambda i: (0, 0)),
            pl.BlockSpec((2 * H, H), lambda i: (0, 0)),
            pl.BlockSpec((1, H), lambda i: (0, 0)),
        ],
        out_specs=pl.BlockSpec((blk, H), lambda i: (i, 0)),
        out_shape=jax.ShapeDtypeStruct((N_NODES, H), jnp.float32),
    )(x, agg2, agg2, w1x, w1a, b1, w2, b2)


def kernel(x, x_s, edge_index, edge_features,
           att_W1, att_b1, att_W2, att_b2,
           upd_W1, upd_b1, upd_W2, upd_b2):
    src = edge_index[0]
    dst = edge_index[1]

    W1s = att_W1[:, :H]
    W1d = att_W1[:, H:2 * H]
    W1ss = att_W1[:, 2 * H:2 * H + S]
    W1ds = att_W1[:, 2 * H + S:2 * H + 2 * S]
    W1e = att_W1[:, 2 * H + 2 * S:]

    wsT = jnp.pad(jnp.concatenate([W1s, W1ss], axis=1).T, ((0, 0), (0, DPG - W_ATT)))
    wdT = jnp.pad(jnp.concatenate([W1d, W1ds], axis=1).T, ((0, 0), (0, DPG - W_ATT)))
    weT = jnp.pad(W1e.T, ((0, 0), (0, DPE - W_ATT)))
    b1p = jnp.pad(att_b1, (0, DPG - W_ATT))[None, :]
    w2pad = jnp.concatenate([
        jnp.pad(att_W2[0], (0, DPE - W_ATT)),
        jnp.pad(att_b2, (0, L - 1)),
    ])

    xcat = jnp.concatenate([x, x_s], axis=1)
    a_src_p, a_dst_p = _node_prep(xcat, wsT, wdT, b1p)
    e_p = _edge_prep(edge_features, weT)
    exp_sc, denom_part = _sc_phase1(a_src_p, a_dst_p, e_p, src, dst, w2pad)
    agg2 = _sc_phase2(x, exp_sc, denom_part.reshape(NW, DENOM_STRIDE), src, dst)

    return _update_mlp(x, agg2, upd_W1, upd_b1, upd_W2, upd_b2)


# speedup vs baseline: 1.0034x; 1.0034x over previous
import functools

import jax
import jax.numpy as jnp
from jax import lax
from jax.experimental import pallas as pl
from jax.experimental.pallas import tpu as pltpu
from jax.experimental.pallas import tpu_sc as plsc

N_NODES = 10000
N_EDGES = 320000
H = 128
S = 16
F = 16
W_ATT = 272
DPG = 384
DPE = 288
L = 16
NW = 32
CHUNK = 128
SUB = 32
NSUB = CHUNK // SUB
NCH_TOTAL = N_EDGES // CHUNK
DENOM_STRIDE = 10240
STRIPE = 1000

_mesh = plsc.VectorSubcoreMesh(core_axis_name="c", subcore_axis_name="s")
_sc_params = pltpu.CompilerParams(needs_layout_passes=False)


def _node_prep_body(xc_ref, ws_ref, wd_ref, b1_ref, as_ref, ad_ref):
    xc = xc_ref[...]
    as_ref[...] = jnp.dot(xc, ws_ref[...], preferred_element_type=jnp.float32)
    ad_ref[...] = jnp.dot(
        xc, wd_ref[...], preferred_element_type=jnp.float32) + b1_ref[...]


def _node_prep(xcat, wsT, wdT, b1p):
    blk = 2000
    grid = N_NODES // blk
    return pl.pallas_call(
        _node_prep_body,
        grid=(grid,),
        in_specs=[
            pl.BlockSpec((blk, H + S), lambda i: (i, 0)),
            pl.BlockSpec((H + S, DPG), lambda i: (0, 0)),
            pl.BlockSpec((H + S, DPG), lambda i: (0, 0)),
            pl.BlockSpec((1, DPG), lambda i: (0, 0)),
        ],
        out_specs=[
            pl.BlockSpec((blk, DPG), lambda i: (i, 0)),
            pl.BlockSpec((blk, DPG), lambda i: (i, 0)),
        ],
        out_shape=[
            jax.ShapeDtypeStruct((N_NODES, DPG), jnp.float32),
            jax.ShapeDtypeStruct((N_NODES, DPG), jnp.float32),
        ],
    )(xcat, wsT, wdT, b1p)


def _edge_prep_body(ef_ref, we_ref, e_ref):
    e_ref[...] = jnp.dot(
        ef_ref[...], we_ref[...], preferred_element_type=jnp.float32)


def _edge_prep(ef, weT):
    blk = 8000
    grid = N_EDGES // blk
    return pl.pallas_call(
        _edge_prep_body,
        grid=(grid,),
        in_specs=[
            pl.BlockSpec((blk, F), lambda i: (i, 0)),
            pl.BlockSpec((F, DPE), lambda i: (0, 0)),
        ],
        out_specs=pl.BlockSpec((blk, DPE), lambda i: (i, 0)),
        out_shape=jax.ShapeDtypeStruct((N_EDGES, DPE), jnp.float32),
    )(ef, weT)


@functools.partial(
    pl.kernel,
    out_type=[
        jax.ShapeDtypeStruct((N_EDGES,), jnp.float32),
        jax.ShapeDtypeStruct((NW * DENOM_STRIDE,), jnp.float32),
    ],
    mesh=_mesh,
    compiler_params=_sc_params,
    scratch_types=[
        pltpu.VMEM((CHUNK,), jnp.int32),
        pltpu.VMEM((CHUNK,), jnp.int32),
        pltpu.VMEM((CHUNK,), jnp.int32),
        pltpu.VMEM((CHUNK,), jnp.int32),
        [pltpu.VMEM((SUB,), jnp.int32) for _ in range(2)],
        [pltpu.VMEM((SUB,), jnp.int32) for _ in range(2)],
        pltpu.VMEM((2, SUB, DPG), jnp.float32),
        pltpu.VMEM((2, SUB, DPG), jnp.float32),
        pltpu.VMEM((2, SUB, DPE), jnp.float32),
        pltpu.VMEM((CHUNK,), jnp.float32),
        pltpu.VMEM((DENOM_STRIDE,), jnp.float32),
        pltpu.VMEM((DPE + L,), jnp.float32),
        [pltpu.SemaphoreType.DMA for _ in range(2)],
        [pltpu.SemaphoreType.DMA for _ in range(2)],
    ],
)
def _sc_phase1(asrc_hbm, adst_hbm, e_hbm, src_hbm, dst_hbm, w2_hbm,
               exp_out, denom_out,
               srcb0, dstb0, srcb1, dstb1, srcsub, dstsub,
               abufs, abufd, ebuf, expb, denom, w2v, sg, se):
    wid = lax.axis_index("s") * 2 + lax.axis_index("c")
    cnt = 78 + jnp.where(wid < 4, 1, 0)
    start = 78 * wid + jnp.minimum(wid, 4)

    pltpu.sync_copy(w2_hbm, w2v)
    b2s = jnp.sum(w2v[pl.ds(DPE, L)])
    iota16 = lax.iota(jnp.int32, L)
    w2r = [w2v[pl.ds(k * L, L)] for k in range(W_ATT // L + 1)]

    def zero_body(i, _):
        denom[pl.ds(i * L, L)] = jnp.zeros((L,), jnp.float32)
        return 0
    lax.fori_loop(0, DENOM_STRIDE // L, zero_body, 0)

    def stage_idx(b, s_dyn, from_src, from_dst):
        for i in range(SUB // L):
            sl = pl.ds(i * L, L)
            srcsub[b][sl] = from_src[pl.ds(s_dyn * SUB + i * L, L)]
            dstsub[b][sl] = from_dst[pl.ds(s_dyn * SUB + i * L, L)]

    def issue_sub(b, c_dyn, s_dyn):
        base = (start + c_dyn) * CHUNK
        pltpu.async_copy(asrc_hbm.at[srcsub[b]], abufs.at[b], sg[b])
        pltpu.async_copy(adst_hbm.at[dstsub[b]], abufd.at[b], sg[b])
        pltpu.async_copy(e_hbm.at[pl.ds(base + s_dyn * SUB, SUB)], ebuf.at[b],
                         se[b])

    base0 = start * CHUNK
    pltpu.sync_copy(src_hbm.at[pl.ds(base0, CHUNK)], srcb0)
    pltpu.sync_copy(dst_hbm.at[pl.ds(base0, CHUNK)], dstb0)
    for s_ in range(2):
        stage_idx(s_, s_, srcb0, dstb0)
        issue_sub(s_, 0, s_)

    def chunk_body(c, _):
        base = (start + c) * CHUNK
        nxt = c + 1
        has_next = nxt < cnt
        np_odd = (nxt % 2) == 1
        cur_even = (c % 2) == 0

        @pl.when(has_next & np_odd)
        def _():
            nbase = (start + nxt) * CHUNK
            pltpu.sync_copy(src_hbm.at[pl.ds(nbase, CHUNK)], srcb1)
            pltpu.sync_copy(dst_hbm.at[pl.ds(nbase, CHUNK)], dstb1)

        @pl.when(has_next & jnp.logical_not(np_odd))
        def _():
            nbase = (start + nxt) * CHUNK
            pltpu.sync_copy(src_hbm.at[pl.ds(nbase, CHUNK)], srcb0)
            pltpu.sync_copy(dst_hbm.at[pl.ds(nbase, CHUNK)], dstb0)

        for s_ in range(4):
            b = s_ % 2
            pltpu.make_async_copy(asrc_hbm.at[srcsub[b]], abufs.at[b],
                                  sg[b]).wait()
            pltpu.make_async_copy(adst_hbm.at[dstsub[b]], abufd.at[b],
                                  sg[b]).wait()
            pltpu.make_async_copy(e_hbm.at[pl.ds(base + s_ * SUB, SUB)],
                                  ebuf.at[b], se[b]).wait()

            def group_body(g, _):
                def edge_body(j, raw16):
                    e = g * L + j
                    acc = jnp.zeros((L,), jnp.float32)
                    for k in range(W_ATT // L + 1):
                        sl = pl.ds(k * L, L)
                        v = abufs[b, e, sl] + abufd[b, e, sl] + ebuf[b, e, sl]
                        v = jnp.maximum(v, 0.0)
                        acc = acc + v * w2r[k]
                    r = jnp.sum(acc)
                    return jnp.where(iota16 == j, r, raw16)

                raw16 = lax.fori_loop(0, L, edge_body,
                                      jnp.zeros((L,), jnp.float32))
                raw16 = raw16 + b2s
                raw16 = jnp.where(raw16 >= 0.0, raw16, 0.01 * raw16)
                ex16 = jnp.exp(raw16)
                expb[pl.ds(s_ * SUB + g * L, L)] = ex16

                @pl.when(cur_even)
                def _():
                    plsc.addupdate_scatter(
                        denom, [dstb0[pl.ds(s_ * SUB + g * L, L)]], ex16)

                @pl.when(jnp.logical_not(cur_even))
                def _():
                    plsc.addupdate_scatter(
                        denom, [dstb1[pl.ds(s_ * SUB + g * L, L)]], ex16)
                return 0

            lax.fori_loop(0, SUB // L, group_body, 0)

            if s_ < 2:
                @pl.when(cur_even)
                def _():
                    stage_idx(b, s_ + 2, srcb0, dstb0)

                @pl.when(jnp.logical_not(cur_even))
                def _():
                    stage_idx(b, s_ + 2, srcb1, dstb1)
                issue_sub(b, c, s_ + 2)
            else:
                @pl.when(has_next & np_odd)
                def _():
                    stage_idx(b, s_ - 2, srcb1, dstb1)
                    issue_sub(b, nxt, s_ - 2)

                @pl.when(has_next & jnp.logical_not(np_odd))
                def _():
                    stage_idx(b, s_ - 2, srcb0, dstb0)
                    issue_sub(b, nxt, s_ - 2)

        pltpu.sync_copy(expb, exp_out.at[pl.ds(base, CHUNK)])
        return 0

    lax.fori_loop(0, cnt, chunk_body, 0)
    pltpu.sync_copy(denom, denom_out.at[pl.ds(wid * DENOM_STRIDE, DENOM_STRIDE)])


@functools.partial(
    pl.kernel,
    out_type=jax.ShapeDtypeStruct((2 * N_NODES, H), jnp.float32),
    mesh=_mesh,
    compiler_params=_sc_params,
    scratch_types=[
        [pltpu.VMEM((CHUNK,), jnp.int32) for _ in range(2)],
        [pltpu.VMEM((CHUNK,), jnp.int32) for _ in range(2)],
        [pltpu.VMEM((CHUNK,), jnp.float32) for _ in range(2)],
        pltpu.VMEM((CHUNK,), jnp.float32),
        pltpu.VMEM((2, CHUNK, H), jnp.float32),
        pltpu.VMEM((DENOM_STRIDE,), jnp.float32),
        pltpu.VMEM((40, H), jnp.float32),
        pltpu.VMEM_SHARED((DENOM_STRIDE,), jnp.float32),
        pltpu.VMEM_SHARED((N_NODES, H), jnp.float32),
        [pltpu.SemaphoreType.DMA for _ in range(2)],
        [pltpu.SemaphoreType.DMA for _ in range(2)],
    ],
)
def _sc_phase2(x_hbm, exp_hbm, denom_part_hbm, src_hbm, dst_hbm,
               agg_out,
               srcb, dstb, expb, attnb, xbuf, denom, tz, denom_sh,
               agg_sp, sg, ss):
    cid = lax.axis_index("c")
    sid = lax.axis_index("s")
    wid = sid * 2 + cid
    cnt = 78 + jnp.where(wid < 4, 1, 0)
    start = 78 * wid + jnp.minimum(wid, 4)

    stripe0 = sid * (DENOM_STRIDE // 16)
    for q in range(DENOM_STRIDE // 16 // 128):
        pltpu.sync_copy(
            denom_part_hbm.at[:, pl.ds(stripe0 + q * 128, 128)],
            tz.at[pl.ds(0, NW)])

        def red_body(i, _):
            sl = pl.ds(i * L, L)
            acc = tz[0, sl]
            for p in range(1, NW):
                acc = acc + tz[p, sl]
            denom[pl.ds(stripe0 + q * 128 + i * L, L)] = acc
            return 0
        lax.fori_loop(0, 128 // L, red_body, 0)
    pltpu.sync_copy(denom.at[pl.ds(stripe0, DENOM_STRIDE // 16)],
                    denom_sh.at[pl.ds(stripe0, DENOM_STRIDE // 16)])

    def zero_body(i, _):
        for k in range(H // L):
            tz[i, pl.ds(k * L, L)] = jnp.zeros((L,), jnp.float32)
        return 0
    lax.fori_loop(0, 40, zero_body, 0)

    @pl.when(sid < 10)
    def _():
        def zcopy_body(r, _):
            pltpu.sync_copy(tz, agg_sp.at[pl.ds(sid * STRIPE + r * 40, 40)])
            return 0
        lax.fori_loop(0, STRIPE // 40, zcopy_body, 0)
    plsc.subcore_barrier()
    pltpu.sync_copy(denom_sh, denom)

    def fetch(b, c_dyn):
        base = (start + c_dyn) * CHUNK
        pltpu.sync_copy(src_hbm.at[pl.ds(base, CHUNK)], srcb[b])
        pltpu.sync_copy(dst_hbm.at[pl.ds(base, CHUNK)], dstb[b])
        pltpu.sync_copy(exp_hbm.at[pl.ds(base, CHUNK)], expb[b])
        pltpu.async_copy(x_hbm.at[srcb[b]], xbuf.at[b], sg[b])

    fetch(0, 0)

    def chunk_body(c, _):
        nxt = c + 1
        for b in range(2):
            @pl.when((c % 2) == b)
            def _():
                pltpu.make_async_copy(x_hbm.at[srcb[b]], xbuf.at[b],
                                      sg[b]).wait()

                def attn_body(g, _):
                    sl = pl.ds(g * L, L)
                    den16 = plsc.load_gather(denom, [dstb[b][sl]])
                    attnb[sl] = expb[b][sl] / (den16 + 1e-09)
                    return 0
                lax.fori_loop(0, CHUNK // L, attn_body, 0)

                def scale_body(g, _):
                    a16 = attnb[pl.ds(g * L, L)]
                    for j in range(L):
                        e = g * L + j
                        aj = jnp.take(a16, jnp.full((L,), j, jnp.int32))
                        for k in range(H // L):
                            sl = pl.ds(k * L, L)
                            xbuf[b, e, sl] = xbuf[b, e, sl] * aj
                    return 0
                lax.fori_loop(0, CHUNK // L, scale_body, 0)

                pltpu.async_copy(xbuf.at[b], agg_sp.at[dstb[b]], ss[b],
                                 add=True)
                @pl.when(nxt < cnt)
                def _():
                    @pl.when(c >= 1)
                    def _():
                        pltpu.make_async_copy(
                            xbuf.at[1 - b], agg_sp.at[dstb[1 - b]],
                            ss[1 - b]).wait()
                    fetch(1 - b, nxt)
        return 0

    lax.fori_loop(0, cnt, chunk_body, 0)
    for b in range(2):
        pltpu.make_async_copy(xbuf.at[b], agg_sp.at[dstb[b]], ss[b]).wait()
    plsc.subcore_barrier()

    @pl.when(sid < 10)
    def _():
        pltpu.sync_copy(
            agg_sp.at[pl.ds(sid * STRIPE, STRIPE)],
            agg_out.at[pl.ds(cid * N_NODES + sid * STRIPE, STRIPE)])


def _update_mlp_body(x_ref, a0_ref, a1_ref, w1x_ref, w1a_ref, b1_ref, w2_ref,
                     b2_ref, out_ref):
    agg = a0_ref[...] + a1_ref[...]
    h = jnp.dot(x_ref[...], w1x_ref[...], preferred_element_type=jnp.float32)
    h += jnp.dot(agg, w1a_ref[...], preferred_element_type=jnp.float32)
    h = jax.nn.relu(h + b1_ref[...])
    o = jnp.dot(h, w2_ref[...], preferred_element_type=jnp.float32) + b2_ref[...]
    out_ref[...] = jax.nn.relu(o)


def _update_mlp(x, agg2, upd_W1, upd_b1, upd_W2, upd_b2):
    blk = 2000
    grid = N_NODES // blk
    w1x = upd_W1[:, :H].T
    w1a = upd_W1[:, H:].T
    w2 = upd_W2.T
    b1 = upd_b1[None, :]
    b2 = upd_b2[None, :]
    return pl.pallas_call(
        _update_mlp_body,
        grid=(grid,),
        in_specs=[
            pl.BlockSpec((blk, H), lambda i: (i, 0)),
            pl.BlockSpec((blk, H), lambda i: (i, 0)),
            pl.BlockSpec((blk, H), lambda i: (i + grid, 0)),
            pl.BlockSpec((H, 2 * H), lambda i: (0, 0)),
            pl.BlockSpec((H, 2 * H), lambda i: (0, 0)),
            pl.BlockSpec((1, 2 * H), lambda i: (0, 0)),
            pl.BlockSpec((2 * H, H), lambda i: (0, 0)),
            pl.BlockSpec((1, H), lambda i: (0, 0)),
        ],
        out_specs=pl.BlockSpec((blk, H), lambda i: (i, 0)),
        out_shape=jax.ShapeDtypeStruct((N_NODES, H), jnp.float32),
    )(x, agg2, agg2, w1x, w1a, b1, w2, b2)


def kernel(x, x_s, edge_index, edge_features,
           att_W1, att_b1, att_W2, att_b2,
           upd_W1, upd_b1, upd_W2, upd_b2):
    src = edge_index[0]
    dst = edge_index[1]

    W1s = att_W1[:, :H]
    W1d = att_W1[:, H:2 * H]
    W1ss = att_W1[:, 2 * H:2 * H + S]
    W1ds = att_W1[:, 2 * H + S:2 * H + 2 * S]
    W1e = att_W1[:, 2 * H + 2 * S:]

    wsT = jnp.pad(jnp.concatenate([W1s, W1ss], axis=1).T, ((0, 0), (0, DPG - W_ATT)))
    wdT = jnp.pad(jnp.concatenate([W1d, W1ds], axis=1).T, ((0, 0), (0, DPG - W_ATT)))
    weT = jnp.pad(W1e.T, ((0, 0), (0, DPE - W_ATT)))
    b1p = jnp.pad(att_b1, (0, DPG - W_ATT))[None, :]
    w2pad = jnp.concatenate([
        jnp.pad(att_W2[0], (0, DPE - W_ATT)),
        jnp.pad(att_b2, (0, L - 1)),
    ])

    xcat = jnp.concatenate([x, x_s], axis=1)
    a_src_p, a_dst_p = _node_prep(xcat, wsT, wdT, b1p)
    e_p = _edge_prep(edge_features, weT)
    exp_sc, denom_part = _sc_phase1(a_src_p, a_dst_p, e_p, src, dst, w2pad)
    agg2 = _sc_phase2(x, exp_sc, denom_part.reshape(NW, DENOM_STRIDE), src, dst)

    return _update_mlp(x, agg2, upd_W1, upd_b1, upd_W2, upd_b2)
